# Initial kernel scaffold; baseline (speedup 1.0000x reference)
import jax
import jax.numpy as jnp
from jax.experimental import pallas as pl


def kernel(input, embed):
    raise NotImplementedError("write your pallas kernel here")


# speedup vs baseline: 1.0718x; 1.0718x over previous
import jax
import jax.numpy as jnp
from jax import lax
from jax.experimental import pallas as pl
from jax.experimental.pallas import tpu as pltpu
from jax.experimental.pallas import tpu_sc as plsc

DIM = 32
N_EMBED = 8192
N_ROWS = 8192
ROW_BLOCK = 1024
CODE_CHUNK = 2048
N_CHUNKS = N_EMBED // CODE_CHUNK
GRID = N_ROWS // ROW_BLOCK


def _tc_body(x_ref, e_ref, x2_ref, e2_ref, ind_ref):
    x2 = x2_ref[...]
    x_bf = x_ref[...].astype(jnp.bfloat16)
    e_bf = e_ref[...].astype(jnp.bfloat16)

    best_i = None
    acc = None
    for c in range(N_CHUNKS):
        c0 = c * CODE_CHUNK
        ec = e_bf[:, c0:c0 + CODE_CHUNK]
        m = lax.dot_general(
            x_bf, ec, (((1,), (0,)), ((), ())),
            preferred_element_type=jnp.float32)
        d = (x2 - 2.0 * m) + e2_ref[:, c0:c0 + CODE_CHUNK]
        mv = jnp.min(d, axis=1, keepdims=True)
        iota = lax.broadcasted_iota(jnp.int32, d.shape, 1)
        mi = jnp.min(jnp.where(d == mv, iota, jnp.int32(2**30)), axis=1,
                     keepdims=True) + jnp.int32(c0)
        neg = -mv
        neg_r = neg.astype(jnp.bfloat16).astype(jnp.float32)
        if best_i is None:
            best_i, acc = mi, neg_r
        else:
            take = neg > acc
            best_i = jnp.where(take, mi, best_i)
            acc = jnp.where(take, neg_r, acc)

    ind_ref[...] = best_i


def _tc_argmin(flatten, embed, x2, e2):
    return pl.pallas_call(
        _tc_body,
        grid=(GRID,),
        in_specs=[
            pl.BlockSpec((ROW_BLOCK, DIM), lambda i: (i, 0)),
            pl.BlockSpec((DIM, N_EMBED), lambda i: (0, 0)),
            pl.BlockSpec((ROW_BLOCK, 1), lambda i: (i, 0)),
            pl.BlockSpec((1, N_EMBED), lambda i: (0, 0)),
        ],
        out_specs=pl.BlockSpec((ROW_BLOCK, 1), lambda i: (i, 0)),
        out_shape=jax.ShapeDtypeStruct((N_ROWS, 1), jnp.int32),
    )(flatten, embed, x2, e2)


def _sc_gather_body(table_ref, idx_ref, out_ref, idx_va, idx_vb,
                    rows_va, rows_vb, sem_a, sem_b):
    wid = lax.axis_index("s") * 2 + lax.axis_index("c")
    base = wid * 256
    pltpu.sync_copy(idx_ref.at[pl.ds(base, 128)], idx_va)
    pltpu.sync_copy(idx_ref.at[pl.ds(base + 128, 128)], idx_vb)
    cp_a = pltpu.async_copy(table_ref.at[idx_va], rows_va, sem_a)
    cp_b = pltpu.async_copy(table_ref.at[idx_vb], rows_vb, sem_b)
    cp_a.wait()
    cp_b.wait()
    pltpu.sync_copy(rows_va, out_ref.at[pl.ds(base, 128)])
    pltpu.sync_copy(rows_vb, out_ref.at[pl.ds(base + 128, 128)])


def _sc_gather(table, idx_flat):
    mesh = plsc.VectorSubcoreMesh(core_axis_name="c", subcore_axis_name="s")
    k = pl.kernel(
        _sc_gather_body,
        out_type=jax.ShapeDtypeStruct((N_ROWS, DIM), jnp.float32),
        mesh=mesh,
        scratch_types=[
            pltpu.VMEM((128,), jnp.int32),
            pltpu.VMEM((128,), jnp.int32),
            pltpu.VMEM((128, DIM), jnp.float32),
            pltpu.VMEM((128, DIM), jnp.float32),
            pltpu.SemaphoreType.DMA,
            pltpu.SemaphoreType.DMA,
        ],
        compiler_params=pltpu.CompilerParams(use_tc_tiling_on_sc=False),
    )
    return k(table, idx_flat)


def _diff_body(q_ref, x_ref, out_ref):
    r = q_ref[...] - x_ref[...]
    out_ref[0, 0] = jnp.sum(r * r)


def _tc_diff(q, flatten):
    return pl.pallas_call(
        _diff_body,
        in_specs=[
            pl.BlockSpec((N_ROWS, DIM), lambda: (0, 0)),
            pl.BlockSpec((N_ROWS, DIM), lambda: (0, 0)),
        ],
        out_specs=pl.BlockSpec(block_shape=(1, 1), index_map=lambda: (0, 0),
                               memory_space=pltpu.SMEM),
        out_shape=jax.ShapeDtypeStruct((1, 1), jnp.float32),
    )(q, flatten)


def kernel(input, embed):
    flatten = input.reshape(-1, DIM)
    x2 = jnp.sum(flatten ** 2, axis=1, keepdims=True)
    e2 = jnp.sum(embed ** 2, axis=0, keepdims=True)
    ind_col = _tc_argmin(flatten, embed, x2, e2)
    ind_flat = ind_col.reshape(N_ROWS)
    embed_ind = ind_flat.reshape(input.shape[:-1])
    table = embed.T
    q = _sc_gather(table, ind_flat)
    quantize_st = q.reshape(input.shape)
    diff_raw = _tc_diff(q, flatten)
    diff = (diff_raw[0, 0] * (1.0 / (N_ROWS * DIM))).reshape(())
    return (quantize_st, diff, embed_ind)


# speedup vs baseline: 1.2711x; 1.1859x over previous
import jax
import jax.numpy as jnp
from jax import lax
from jax.experimental import pallas as pl
from jax.experimental.pallas import tpu as pltpu
from jax.experimental.pallas import tpu_sc as plsc

DIM = 32
N_EMBED = 8192
N_ROWS = 8192
ROW_BLOCK = 1024
CODE_CHUNK = 2048
N_CHUNKS = N_EMBED // CODE_CHUNK
GRID = N_ROWS // ROW_BLOCK


def _tc_body(x_ref, e_ref, x2_ref, e2_ref, ind_ref):
    x2 = x2_ref[...]
    x_bf = (2.0 * x_ref[...]).astype(jnp.bfloat16)
    e_bf = e_ref[...].astype(jnp.bfloat16)
    lane = lax.broadcasted_iota(jnp.int32, (ROW_BLOCK, 128), 1)

    best_i = None
    acc = None
    for c in range(N_CHUNKS):
        c0 = c * CODE_CHUNK
        ec = e_bf[:, c0:c0 + CODE_CHUNK]
        m2 = lax.dot_general(
            x_bf, ec, (((1,), (0,)), ((), ())),
            preferred_element_type=jnp.float32)
        d = (x2 - m2) + e2_ref[:, c0:c0 + CODE_CHUNK]
        runm = d[:, 0:128]
        runk = jnp.zeros((ROW_BLOCK, 128), jnp.int32)
        for k in range(1, CODE_CHUNK // 128):
            v = d[:, k * 128:(k + 1) * 128]
            lt = v < runm
            runm = jnp.where(lt, v, runm)
            runk = jnp.where(lt, jnp.int32(k), runk)
        mv = jnp.min(runm, axis=1, keepdims=True)
        gidx = runk * jnp.int32(128) + lane
        mi = jnp.min(jnp.where(runm == mv, gidx, jnp.int32(2**30)), axis=1,
                     keepdims=True) + jnp.int32(c0)
        neg = -mv
        neg_r = neg.astype(jnp.bfloat16).astype(jnp.float32)
        if best_i is None:
            best_i, acc = mi, neg_r
        else:
            take = neg > acc
            best_i = jnp.where(take, mi, best_i)
            acc = jnp.where(take, neg_r, acc)

    ind_ref[...] = best_i


def _tc_argmin(flatten, embed, x2, e2):
    return pl.pallas_call(
        _tc_body,
        grid=(GRID,),
        in_specs=[
            pl.BlockSpec((ROW_BLOCK, DIM), lambda i: (i, 0)),
            pl.BlockSpec((DIM, N_EMBED), lambda i: (0, 0)),
            pl.BlockSpec((ROW_BLOCK, 1), lambda i: (i, 0)),
            pl.BlockSpec((1, N_EMBED), lambda i: (0, 0)),
        ],
        out_specs=pl.BlockSpec((ROW_BLOCK, 1), lambda i: (i, 0)),
        out_shape=jax.ShapeDtypeStruct((N_ROWS, 1), jnp.int32),
    )(flatten, embed, x2, e2)


def _sc_gather_body(table_ref, idx_ref, out_ref, idx_va, idx_vb,
                    rows_va, rows_vb, sem_a, sem_b):
    wid = lax.axis_index("s") * 2 + lax.axis_index("c")
    base = wid * 256
    pltpu.sync_copy(idx_ref.at[pl.ds(base, 128)], idx_va)
    pltpu.sync_copy(idx_ref.at[pl.ds(base + 128, 128)], idx_vb)
    cp_a = pltpu.async_copy(table_ref.at[idx_va], rows_va, sem_a)
    cp_b = pltpu.async_copy(table_ref.at[idx_vb], rows_vb, sem_b)
    cp_a.wait()
    cp_b.wait()
    pltpu.sync_copy(rows_va, out_ref.at[pl.ds(base, 128)])
    pltpu.sync_copy(rows_vb, out_ref.at[pl.ds(base + 128, 128)])


def _sc_gather(table, idx_flat):
    mesh = plsc.VectorSubcoreMesh(core_axis_name="c", subcore_axis_name="s")
    k = pl.kernel(
        _sc_gather_body,
        out_type=jax.ShapeDtypeStruct((N_ROWS, DIM), jnp.float32),
        mesh=mesh,
        scratch_types=[
            pltpu.VMEM((128,), jnp.int32),
            pltpu.VMEM((128,), jnp.int32),
            pltpu.VMEM((128, DIM), jnp.float32),
            pltpu.VMEM((128, DIM), jnp.float32),
            pltpu.SemaphoreType.DMA,
            pltpu.SemaphoreType.DMA,
        ],
        compiler_params=pltpu.CompilerParams(use_tc_tiling_on_sc=False),
    )
    return k(table, idx_flat)


def _diff_body(q_ref, x_ref, out_ref):
    r = q_ref[...] - x_ref[...]
    out_ref[0, 0] = jnp.sum(r * r)


def _tc_diff(q, flatten):
    return pl.pallas_call(
        _diff_body,
        in_specs=[
            pl.BlockSpec((N_ROWS, DIM), lambda: (0, 0)),
            pl.BlockSpec((N_ROWS, DIM), lambda: (0, 0)),
        ],
        out_specs=pl.BlockSpec(block_shape=(1, 1), index_map=lambda: (0, 0),
                               memory_space=pltpu.SMEM),
        out_shape=jax.ShapeDtypeStruct((1, 1), jnp.float32),
    )(q, flatten)


def kernel(input, embed):
    flatten = input.reshape(-1, DIM)
    x2 = jnp.sum(flatten ** 2, axis=1, keepdims=True)
    e2 = jnp.sum(embed ** 2, axis=0, keepdims=True)
    ind_col = _tc_argmin(flatten, embed, x2, e2)
    ind_flat = ind_col.reshape(N_ROWS)
    embed_ind = ind_flat.reshape(input.shape[:-1])
    table = embed.T
    q = _sc_gather(table, ind_flat)
    quantize_st = q.reshape(input.shape)
    diff_raw = _tc_diff(q, flatten)
    diff = (diff_raw[0, 0] * (1.0 / (N_ROWS * DIM))).reshape(())
    return (quantize_st, diff, embed_ind)


# speedup vs baseline: 1.3499x; 1.0620x over previous
import jax
import jax.numpy as jnp
from jax import lax
from jax.experimental import pallas as pl
from jax.experimental.pallas import tpu as pltpu
from jax.experimental.pallas import tpu_sc as plsc

DIM = 32
N_EMBED = 8192
N_ROWS = 8192
ROW_BLOCK = 1024
CODE_CHUNK = 2048
N_CHUNKS = N_EMBED // CODE_CHUNK
GRID = N_ROWS // ROW_BLOCK


def _tc_body(x_ref, e_ref, x2_ref, e2_ref, ind_ref, dsum_ref):
    x2 = x2_ref[...]
    x_bf = (2.0 * x_ref[...]).astype(jnp.bfloat16)
    e_bf = e_ref[...].astype(jnp.bfloat16)
    lane = lax.broadcasted_iota(jnp.int32, (ROW_BLOCK, 128), 1)

    best_i = None
    acc = None
    for c in range(N_CHUNKS):
        c0 = c * CODE_CHUNK
        ec = e_bf[:, c0:c0 + CODE_CHUNK]
        m2 = lax.dot_general(
            x_bf, ec, (((1,), (0,)), ((), ())),
            preferred_element_type=jnp.float32)
        d = (x2 - m2) + e2_ref[:, c0:c0 + CODE_CHUNK]
        runm = d[:, 0:128]
        runk = jnp.zeros((ROW_BLOCK, 128), jnp.int32)
        for k in range(1, CODE_CHUNK // 128):
            v = d[:, k * 128:(k + 1) * 128]
            lt = v < runm
            runm = jnp.where(lt, v, runm)
            runk = jnp.where(lt, jnp.int32(k), runk)
        mv = jnp.min(runm, axis=1, keepdims=True)
        gidx = runk * jnp.int32(128) + lane
        mi = jnp.min(jnp.where(runm == mv, gidx, jnp.int32(2**30)), axis=1,
                     keepdims=True) + jnp.int32(c0)
        neg = -mv
        neg_r = neg.astype(jnp.bfloat16).astype(jnp.float32)
        if best_i is None:
            best_i, acc, dmin = mi, neg_r, mv
        else:
            take = neg > acc
            best_i = jnp.where(take, mi, best_i)
            acc = jnp.where(take, neg_r, acc)
            dmin = jnp.minimum(dmin, mv)

    ind_ref[...] = best_i
    blk = jnp.sum(dmin)
    @pl.when(pl.program_id(0) == 0)
    def _init():
        dsum_ref[0, 0] = blk
    @pl.when(pl.program_id(0) != 0)
    def _acc():
        dsum_ref[0, 0] = dsum_ref[0, 0] + blk


def _tc_argmin(flatten, embed, x2, e2):
    return pl.pallas_call(
        _tc_body,
        grid=(GRID,),
        in_specs=[
            pl.BlockSpec((ROW_BLOCK, DIM), lambda i: (i, 0)),
            pl.BlockSpec((DIM, N_EMBED), lambda i: (0, 0)),
            pl.BlockSpec((ROW_BLOCK, 1), lambda i: (i, 0)),
            pl.BlockSpec((1, N_EMBED), lambda i: (0, 0)),
        ],
        out_specs=[
            pl.BlockSpec((ROW_BLOCK, 1), lambda i: (i, 0)),
            pl.BlockSpec(block_shape=(1, 1), index_map=lambda i: (0, 0),
                         memory_space=pltpu.SMEM),
        ],
        out_shape=[
            jax.ShapeDtypeStruct((N_ROWS, 1), jnp.int32),
            jax.ShapeDtypeStruct((1, 1), jnp.float32),
        ],
    )(flatten, embed, x2, e2)


def _sc_gather_body(table_ref, idx_ref, out_ref, idx_va, idx_vb,
                    rows_va, rows_vb, sem_a, sem_b):
    wid = lax.axis_index("s") * 2 + lax.axis_index("c")
    base = wid * 256
    pltpu.sync_copy(idx_ref.at[pl.ds(base, 128)], idx_va)
    pltpu.sync_copy(idx_ref.at[pl.ds(base + 128, 128)], idx_vb)
    cp_a = pltpu.async_copy(table_ref.at[idx_va], rows_va, sem_a)
    cp_b = pltpu.async_copy(table_ref.at[idx_vb], rows_vb, sem_b)
    cp_a.wait()
    cp_b.wait()
    pltpu.sync_copy(rows_va, out_ref.at[pl.ds(base, 128)])
    pltpu.sync_copy(rows_vb, out_ref.at[pl.ds(base + 128, 128)])


def _sc_gather(table, idx_flat):
    mesh = plsc.VectorSubcoreMesh(core_axis_name="c", subcore_axis_name="s")
    k = pl.kernel(
        _sc_gather_body,
        out_type=jax.ShapeDtypeStruct((N_ROWS, DIM), jnp.float32),
        mesh=mesh,
        scratch_types=[
            pltpu.VMEM((128,), jnp.int32),
            pltpu.VMEM((128,), jnp.int32),
            pltpu.VMEM((128, DIM), jnp.float32),
            pltpu.VMEM((128, DIM), jnp.float32),
            pltpu.SemaphoreType.DMA,
            pltpu.SemaphoreType.DMA,
        ],
        compiler_params=pltpu.CompilerParams(use_tc_tiling_on_sc=False),
    )
    return k(table, idx_flat)


def kernel(input, embed):
    flatten = input.reshape(-1, DIM)
    x2 = jnp.sum(flatten ** 2, axis=1, keepdims=True)
    e2 = jnp.sum(embed ** 2, axis=0, keepdims=True)
    ind_col, dsum = _tc_argmin(flatten, embed, x2, e2)
    ind_flat = ind_col.reshape(N_ROWS)
    embed_ind = ind_flat.reshape(input.shape[:-1])
    table = embed.T
    q = _sc_gather(table, ind_flat)
    quantize_st = q.reshape(input.shape)
    diff = (dsum[0, 0] * (1.0 / (N_ROWS * DIM))).reshape(())
    return (quantize_st, diff, embed_ind)


# speedup vs baseline: 1.3630x; 1.0097x over previous
import jax
import jax.numpy as jnp
from jax import lax
from jax.experimental import pallas as pl
from jax.experimental.pallas import tpu as pltpu
from jax.experimental.pallas import tpu_sc as plsc

DIM = 32
N_EMBED = 8192
N_ROWS = 8192
ROW_BLOCK = 1024
CODE_CHUNK = 2048
N_CHUNKS = N_EMBED // CODE_CHUNK
GRID = N_ROWS // ROW_BLOCK


def _tc_body(x_ref, e_ref, x2_ref, e2_ref, ind_ref, dsum_ref, et_ref):
    x2 = x2_ref[...]
    x_bf = (2.0 * x_ref[...]).astype(jnp.bfloat16)
    e_bf = e_ref[...].astype(jnp.bfloat16)
    lane = lax.broadcasted_iota(jnp.int32, (ROW_BLOCK, 128), 1)

    best_i = None
    acc = None
    for c in range(N_CHUNKS):
        c0 = c * CODE_CHUNK
        ec = e_bf[:, c0:c0 + CODE_CHUNK]
        m2 = lax.dot_general(
            x_bf, ec, (((1,), (0,)), ((), ())),
            preferred_element_type=jnp.float32)
        d = (x2 - m2) + e2_ref[:, c0:c0 + CODE_CHUNK]
        runm = d[:, 0:128]
        runk = jnp.zeros((ROW_BLOCK, 128), jnp.int32)
        for k in range(1, CODE_CHUNK // 128):
            v = d[:, k * 128:(k + 1) * 128]
            lt = v < runm
            runm = jnp.where(lt, v, runm)
            runk = jnp.where(lt, jnp.int32(k), runk)
        mv = jnp.min(runm, axis=1, keepdims=True)
        gidx = runk * jnp.int32(128) + lane
        mi = jnp.min(jnp.where(runm == mv, gidx, jnp.int32(2**30)), axis=1,
                     keepdims=True) + jnp.int32(c0)
        neg = -mv
        neg_r = neg.astype(jnp.bfloat16).astype(jnp.float32)
        if best_i is None:
            best_i, acc, dmin = mi, neg_r, mv
        else:
            take = neg > acc
            best_i = jnp.where(take, mi, best_i)
            acc = jnp.where(take, neg_r, acc)
            dmin = jnp.minimum(dmin, mv)

    ind_ref[...] = best_i
    blk = jnp.sum(dmin) * (1.0 / (N_ROWS * DIM))
    @pl.when(pl.program_id(0) == 0)
    def _init():
        dsum_ref[0, 0] = blk
        et_ref[...] = e_ref[...].T
    @pl.when(pl.program_id(0) != 0)
    def _acc():
        dsum_ref[0, 0] = dsum_ref[0, 0] + blk


def _tc_argmin(flatten, embed, x2, e2):
    return pl.pallas_call(
        _tc_body,
        grid=(GRID,),
        in_specs=[
            pl.BlockSpec((ROW_BLOCK, DIM), lambda i: (i, 0)),
            pl.BlockSpec((DIM, N_EMBED), lambda i: (0, 0)),
            pl.BlockSpec((ROW_BLOCK, 1), lambda i: (i, 0)),
            pl.BlockSpec((1, N_EMBED), lambda i: (0, 0)),
        ],
        out_specs=[
            pl.BlockSpec((ROW_BLOCK, 1), lambda i: (i, 0)),
            pl.BlockSpec(block_shape=(1, 1), index_map=lambda i: (0, 0),
                         memory_space=pltpu.SMEM),
            pl.BlockSpec((N_EMBED, DIM), lambda i: (0, 0)),
        ],
        out_shape=[
            jax.ShapeDtypeStruct((N_ROWS, 1), jnp.int32),
            jax.ShapeDtypeStruct((1, 1), jnp.float32),
            jax.ShapeDtypeStruct((N_EMBED, DIM), jnp.float32),
        ],
    )(flatten, embed, x2, e2)


def _sc_gather_body(table_ref, idx_ref, out_ref, idx_v, rows_v, sem):
    wid = lax.axis_index("s") * 2 + lax.axis_index("c")
    base = wid * 256
    pltpu.sync_copy(idx_ref.at[pl.ds(base, 256)], idx_v)
    cp = pltpu.async_copy(table_ref.at[idx_v], rows_v, sem)
    cp.wait()
    pltpu.sync_copy(rows_v, out_ref.at[pl.ds(base, 256)])


def _sc_gather(table, idx_flat):
    mesh = plsc.VectorSubcoreMesh(core_axis_name="c", subcore_axis_name="s")
    k = pl.kernel(
        _sc_gather_body,
        out_type=jax.ShapeDtypeStruct((N_ROWS, DIM), jnp.float32),
        mesh=mesh,
        scratch_types=[
            pltpu.VMEM((256,), jnp.int32),
            pltpu.VMEM((256, DIM), jnp.float32),
            pltpu.SemaphoreType.DMA,
        ],
        compiler_params=pltpu.CompilerParams(use_tc_tiling_on_sc=False),
    )
    return k(table, idx_flat)


def kernel(input, embed):
    flatten = input.reshape(-1, DIM)
    x2 = jnp.sum(flatten ** 2, axis=1, keepdims=True)
    e2 = jnp.sum(embed ** 2, axis=0, keepdims=True)
    ind_col, dsum, table = _tc_argmin(flatten, embed, x2, e2)
    ind_flat = ind_col.reshape(N_ROWS)
    embed_ind = ind_flat.reshape(input.shape[:-1])
    q = _sc_gather(table, ind_flat)
    quantize_st = q.reshape(input.shape)
    diff = dsum.reshape(())
    return (quantize_st, diff, embed_ind)


# speedup vs baseline: 1.3793x; 1.0120x over previous
import jax
import jax.numpy as jnp
from jax import lax
from jax.experimental import pallas as pl
from jax.experimental.pallas import tpu as pltpu
from jax.experimental.pallas import tpu_sc as plsc

DIM = 32
N_EMBED = 8192
N_ROWS = 8192
ROW_BLOCK = 1024
CODE_CHUNK = 2048
N_CHUNKS = N_EMBED // CODE_CHUNK
GRID = N_ROWS // ROW_BLOCK


def _tc_body(x_ref, e_ref, x2_ref, e2_ref, ind_ref, dsum_ref, et_ref):
    x2 = x2_ref[...]
    x_bf = (2.0 * x_ref[...]).astype(jnp.bfloat16)
    e_bf = e_ref[...].astype(jnp.bfloat16)
    lane = lax.broadcasted_iota(jnp.int32, (ROW_BLOCK, 128), 1)

    best_i = None
    acc = None
    for c in range(N_CHUNKS):
        c0 = c * CODE_CHUNK
        ec = e_bf[:, c0:c0 + CODE_CHUNK]
        m2 = lax.dot_general(
            x_bf, ec, (((1,), (0,)), ((), ())),
            preferred_element_type=jnp.float32)
        d = (x2 - m2) + e2_ref[:, c0:c0 + CODE_CHUNK]
        runm = d[:, 0:128]
        runk = jnp.zeros((ROW_BLOCK, 128), jnp.int32)
        for k in range(1, CODE_CHUNK // 128):
            v = d[:, k * 128:(k + 1) * 128]
            lt = v < runm
            runm = jnp.where(lt, v, runm)
            runk = jnp.where(lt, jnp.int32(k), runk)
        mv = jnp.min(runm, axis=1, keepdims=True)
        gidx = runk * jnp.int32(128) + lane
        mi = jnp.min(jnp.where(runm == mv, gidx, jnp.int32(2**30)), axis=1,
                     keepdims=True) + jnp.int32(c0)
        neg = -mv
        neg_r = neg.astype(jnp.bfloat16).astype(jnp.float32)
        if best_i is None:
            best_i, acc, dmin = mi, neg_r, mv
        else:
            take = neg > acc
            best_i = jnp.where(take, mi, best_i)
            acc = jnp.where(take, neg_r, acc)
            dmin = jnp.minimum(dmin, mv)

    ind_ref[...] = best_i
    dsum_ref[0, 0, 0] = jnp.sum(dmin) * (1.0 / (N_ROWS * DIM))
    i = pl.program_id(0)
    et_ref[...] = e_ref[:, pl.ds(i * (N_EMBED // GRID), N_EMBED // GRID)].T


def _tc_argmin(flatten, embed, x2, e2):
    return pl.pallas_call(
        _tc_body,
        grid=(GRID,),
        in_specs=[
            pl.BlockSpec((ROW_BLOCK, DIM), lambda i: (i, 0)),
            pl.BlockSpec((DIM, N_EMBED), lambda i: (0, 0)),
            pl.BlockSpec((ROW_BLOCK, 1), lambda i: (i, 0)),
            pl.BlockSpec((1, N_EMBED), lambda i: (0, 0)),
        ],
        out_specs=[
            pl.BlockSpec((ROW_BLOCK, 1), lambda i: (i, 0)),
            pl.BlockSpec(block_shape=(1, 1, 1), index_map=lambda i: (i, 0, 0),
                         memory_space=pltpu.SMEM),
            pl.BlockSpec((N_EMBED // GRID, DIM), lambda i: (i, 0)),
        ],
        out_shape=[
            jax.ShapeDtypeStruct((N_ROWS, 1), jnp.int32),
            jax.ShapeDtypeStruct((GRID, 1, 1), jnp.float32),
            jax.ShapeDtypeStruct((N_EMBED, DIM), jnp.float32),
        ],
        compiler_params=pltpu.CompilerParams(
            dimension_semantics=("parallel",)),
    )(flatten, embed, x2, e2)


def _sc_gather_body(table_ref, idx_ref, out_ref, idx_v, rows_v, sem):
    wid = lax.axis_index("s") * 2 + lax.axis_index("c")
    base = wid * 256
    pltpu.sync_copy(idx_ref.at[pl.ds(base, 256)], idx_v)
    cp = pltpu.async_copy(table_ref.at[idx_v], rows_v, sem)
    cp.wait()
    pltpu.sync_copy(rows_v, out_ref.at[pl.ds(base, 256)])


def _sc_gather(table, idx_flat):
    mesh = plsc.VectorSubcoreMesh(core_axis_name="c", subcore_axis_name="s")
    k = pl.kernel(
        _sc_gather_body,
        out_type=jax.ShapeDtypeStruct((N_ROWS, DIM), jnp.float32),
        mesh=mesh,
        scratch_types=[
            pltpu.VMEM((256,), jnp.int32),
            pltpu.VMEM((256, DIM), jnp.float32),
            pltpu.SemaphoreType.DMA,
        ],
        compiler_params=pltpu.CompilerParams(use_tc_tiling_on_sc=False),
    )
    return k(table, idx_flat)


def kernel(input, embed):
    flatten = input.reshape(-1, DIM)
    x2 = jnp.sum(flatten ** 2, axis=1, keepdims=True)
    e2 = jnp.sum(embed ** 2, axis=0, keepdims=True)
    ind_col, dsum, table = _tc_argmin(flatten, embed, x2, e2)
    ind_flat = ind_col.reshape(N_ROWS)
    embed_ind = ind_flat.reshape(input.shape[:-1])
    q = _sc_gather(table, ind_flat)
    quantize_st = q.reshape(input.shape)
    diff = jnp.sum(dsum).reshape(())
    return (quantize_st, diff, embed_ind)
